# Initial kernel scaffold; baseline (speedup 1.0000x reference)
import jax
import jax.numpy as jnp
from jax.experimental import pallas as pl


def kernel(edge_index, user_emb, item_emb):
    raise NotImplementedError("write your pallas kernel here")


# speedup vs baseline: 10.0149x; 10.0149x over previous
import functools

import jax
import jax.numpy as jnp
from jax import lax
from jax.experimental import pallas as pl
from jax.experimental.pallas import tpu as pltpu
from jax.experimental.pallas import tpu_sc as plsc

N_USERS = 10000
N_ITEMS = 40000
N_NODES = 50000
D = 64
E = 800000
N_LAYERS = 3

NC = 2
NS = 16
HALF = N_NODES // NC
ACC_ROWS = NS * 1568
T_ROWS = 51200
DEG_N = NS * 3200

CHB = 256
EPT = 50176
E_PAD = NS * EPT
DEG_CHB = 512
EPW = E_PAD // (NC * NS)

_mesh = plsc.VectorSubcoreMesh(core_axis_name="c", subcore_axis_name="s")


@functools.partial(
    pl.kernel,
    out_type=jax.ShapeDtypeStruct((2 * DEG_N,), jnp.float32),
    mesh=_mesh,
    compiler_params=pltpu.CompilerParams(use_tc_tiling_on_sc=False),
    scratch_types=[
        pltpu.VMEM((DEG_CHB,), jnp.int32),
        pltpu.VMEM((DEG_CHB,), jnp.float32),
        pltpu.VMEM((DEG_N // NS,), jnp.float32),
        pltpu.VMEM_SHARED((DEG_N,), jnp.float32),
    ],
)
def _deg_kernel(row_hbm, out_hbm, idx_v, ones_v, zv, deg_sh):
    c = lax.axis_index("c")
    s = lax.axis_index("s")
    wid = c * NS + s
    slab = DEG_N // NS
    for i in range(DEG_CHB // 16):
        ones_v[pl.ds(i * 16, 16)] = jnp.full((16,), 1.0, jnp.float32)

    def zbody(i, carry):
        zv[pl.ds(i * 16, 16)] = jnp.zeros((16,), jnp.float32)
        return carry

    lax.fori_loop(0, slab // 16, zbody, 0)
    pltpu.sync_copy(zv, deg_sh.at[pl.ds(s * slab, slab)])
    plsc.subcore_barrier()

    def body(k, carry):
        base = wid * EPW + k * DEG_CHB
        pltpu.sync_copy(row_hbm.at[pl.ds(base, DEG_CHB)], idx_v)
        pltpu.sync_copy(ones_v, deg_sh.at[idx_v], add=True)
        return carry

    lax.fori_loop(0, EPW // DEG_CHB, body, 0)
    plsc.subcore_barrier()
    pltpu.sync_copy(deg_sh.at[pl.ds(s * slab, slab)], zv)
    pltpu.sync_copy(zv, out_hbm.at[pl.ds(c * DEG_N + s * slab, slab)])


@functools.partial(
    pl.kernel,
    out_type=jax.ShapeDtypeStruct((T_ROWS, D), jnp.float32),
    mesh=_mesh,
    compiler_params=pltpu.CompilerParams(use_tc_tiling_on_sc=False),
    scratch_types=[
        pltpu.VMEM((CHB,), jnp.int32),
        pltpu.VMEM((CHB,), jnp.int32),
        pltpu.VMEM((CHB,), jnp.int32),
        pltpu.VMEM((CHB,), jnp.int32),
        pltpu.VMEM((CHB, D), jnp.float32),
        pltpu.VMEM_SHARED((ACC_ROWS, D), jnp.float32),
        pltpu.SemaphoreType.DMA,
    ],
)
def _scatter_kernel(t_hbm, row_hbm, col_hbm, s_hbm,
                    row_v, col_v, gidx_v, sidx_v, gbuf, acc, sem):
    c = lax.axis_index("c")
    s = lax.axis_index("s")
    slab = ACC_ROWS // NS
    cbase = c * HALF

    def zero_fill(dst, dst_off, n):
        full, rem = divmod(n, CHB)
        for r in range(full):
            pltpu.sync_copy(gbuf, dst.at[pl.ds(dst_off + r * CHB, CHB)])
        if rem:
            pltpu.sync_copy(gbuf.at[pl.ds(0, rem)],
                            dst.at[pl.ds(dst_off + full * CHB, rem)])

    def zbody(i, carry):
        for j in range(D // 16):
            gbuf[i, pl.ds(j * 16, 16)] = jnp.zeros((16,), jnp.float32)
        return carry

    lax.fori_loop(0, CHB, zbody, 0)
    zero_fill(acc, s * slab, slab)

    @pl.when((c == NC - 1) & (s == NS - 1))
    def _():
        zero_fill(s_hbm, N_NODES, T_ROWS - N_NODES)

    plsc.subcore_barrier()

    def body(k, carry):
        base = s * EPT + k * CHB
        pltpu.sync_copy(row_hbm.at[pl.ds(base, CHB)], row_v)
        pltpu.sync_copy(col_hbm.at[pl.ds(base, CHB)], col_v)
        for i in range(CHB // 16):
            sl = pl.ds(i * 16, 16)
            cv = col_v[sl]
            rv = row_v[sl]
            local = cv - cbase
            inr = (local >= 0) & (local < HALF)
            alt = jnp.where(local < 0, local + HALF, local - HALF)
            sidx_v[sl] = jnp.where(inr, local, alt)
            gidx_v[sl] = jnp.where(inr, rv, N_NODES + (rv & 1023))
        pltpu.async_copy(t_hbm.at[gidx_v], gbuf, sem).wait()
        pltpu.sync_copy(gbuf, acc.at[sidx_v], add=True)
        return carry

    lax.fori_loop(0, EPT // CHB, body, 0)
    plsc.subcore_barrier()

    def copy_rows(src_off, dst_off, n):
        full, rem = divmod(n, CHB)
        for r in range(full):
            pltpu.sync_copy(acc.at[pl.ds(src_off + r * CHB, CHB)], gbuf)
            pltpu.sync_copy(gbuf, s_hbm.at[pl.ds(dst_off + r * CHB, CHB)])
        if rem:
            pltpu.sync_copy(acc.at[pl.ds(src_off + full * CHB, rem)],
                            gbuf.at[pl.ds(0, rem)])
            pltpu.sync_copy(gbuf.at[pl.ds(0, rem)],
                            s_hbm.at[pl.ds(dst_off + full * CHB, rem)])

    @pl.when(s < NS - 1)
    def _():
        copy_rows(s * slab, cbase + s * slab, slab)

    @pl.when(s == NS - 1)
    def _():
        last = HALF - (NS - 1) * slab
        copy_rows((NS - 1) * slab, cbase + (NS - 1) * slab, last)


def _dinv_body(pd_ref, out_ref):
    a = pd_ref[0:DEG_N]
    b = pd_ref[DEG_N:2 * DEG_N]
    deg = a + b
    out_ref[...] = jnp.where(deg > 0.0,
                             lax.rsqrt(jnp.maximum(deg, 1e-30)), 0.0)


def _scale_body(e_ref, d_ref, t_ref):
    t_ref[...] = d_ref[...] * e_ref[...]


def _combine_body(s_ref, d_ref, a_ref, acc_ref, t_ref):
    d = d_ref[...]
    e = d * s_ref[...]
    acc_ref[...] = a_ref[...] + e
    t_ref[...] = d * e


def _final_body(s_ref, d_ref, a_ref, out_ref):
    e = d_ref[...] * s_ref[...]
    out_ref[...] = (a_ref[...] + e) * 0.25


_BLK = 1024
_GRID = T_ROWS // _BLK

_row_spec = pl.BlockSpec((_BLK, D), lambda i: (i, 0))
_col_spec = pl.BlockSpec((_BLK, 1), lambda i: (i, 0))
_f32t = jax.ShapeDtypeStruct((T_ROWS, D), jnp.float32)

_dinv_call = pl.pallas_call(
    _dinv_body,
    out_shape=jax.ShapeDtypeStruct((DEG_N,), jnp.float32),
)

_scale_call = pl.pallas_call(
    _scale_body,
    grid=(_GRID,),
    in_specs=[_row_spec, _col_spec],
    out_specs=_row_spec,
    out_shape=_f32t,
)

_combine_call = pl.pallas_call(
    _combine_body,
    grid=(_GRID,),
    in_specs=[_row_spec, _col_spec, _row_spec],
    out_specs=(_row_spec, _row_spec),
    out_shape=(_f32t, _f32t),
)

_final_call = pl.pallas_call(
    _final_body,
    grid=(_GRID,),
    in_specs=[_row_spec, _col_spec, _row_spec],
    out_specs=_row_spec,
    out_shape=_f32t,
)


def kernel(edge_index, user_emb, item_emb):
    row = edge_index[0].astype(jnp.int32)
    col = edge_index[1].astype(jnp.int32)
    npad = E_PAD - E
    pad_i = jnp.arange(npad, dtype=jnp.int32)
    row_p = jnp.concatenate([row, N_NODES + (pad_i % 1024)])
    col_p = jnp.concatenate([col, pad_i % HALF])

    deg2 = _deg_kernel(row_p)
    dinv = _dinv_call(deg2)

    dinv_col = jnp.concatenate(
        [dinv[:N_NODES], jnp.zeros((T_ROWS - N_NODES,), jnp.float32)]
    ).reshape(T_ROWS, 1)
    e0 = jnp.concatenate(
        [user_emb, item_emb,
         jnp.zeros((T_ROWS - N_NODES, D), jnp.float32)], axis=0)

    t = _scale_call(e0, dinv_col)
    acc = e0
    for layer in range(N_LAYERS):
        sarr = _scatter_kernel(t, row_p, col_p)
        if layer < N_LAYERS - 1:
            acc, t = _combine_call(sarr, dinv_col, acc)
        else:
            acc = _final_call(sarr, dinv_col, acc)
    return acc[:N_NODES]


# speedup vs baseline: 17.1139x; 1.7088x over previous
import functools

import jax
import jax.numpy as jnp
from jax import lax
from jax.experimental import pallas as pl
from jax.experimental.pallas import tpu as pltpu
from jax.experimental.pallas import tpu_sc as plsc

N_USERS = 10000
N_ITEMS = 40000
N_NODES = 50000
D = 64
E = 800000
N_LAYERS = 3

NC = 2
NS = 16
HALF = N_NODES // NC
ACC_ROWS = NS * 1568
T_ROWS = 51200
DEG_N = NS * 3200

CHB = 224
NCH = 224
NCHP = NCH + 2
EPT = NCH * CHB
E_PAD = NS * EPT
DEG_CHB = 512
EPW = E_PAD // (NC * NS)

_mesh = plsc.VectorSubcoreMesh(core_axis_name="c", subcore_axis_name="s")


@functools.partial(
    pl.kernel,
    out_type=jax.ShapeDtypeStruct((2 * DEG_N,), jnp.float32),
    mesh=_mesh,
    compiler_params=pltpu.CompilerParams(use_tc_tiling_on_sc=False),
    scratch_types=[
        pltpu.VMEM((DEG_CHB,), jnp.int32),
        pltpu.VMEM((DEG_CHB,), jnp.float32),
        pltpu.VMEM((DEG_N // NS,), jnp.float32),
        pltpu.VMEM_SHARED((DEG_N,), jnp.float32),
    ],
)
def _deg_kernel(row_hbm, out_hbm, idx_v, ones_v, zv, deg_sh):
    c = lax.axis_index("c")
    s = lax.axis_index("s")
    wid = c * NS + s
    slab = DEG_N // NS
    for i in range(DEG_CHB // 16):
        ones_v[pl.ds(i * 16, 16)] = jnp.full((16,), 1.0, jnp.float32)

    def zbody(i, carry):
        zv[pl.ds(i * 16, 16)] = jnp.zeros((16,), jnp.float32)
        return carry

    lax.fori_loop(0, slab // 16, zbody, 0)
    pltpu.sync_copy(zv, deg_sh.at[pl.ds(s * slab, slab)])
    plsc.subcore_barrier()

    def body(k, carry):
        base = wid * EPW + k * DEG_CHB
        pltpu.sync_copy(row_hbm.at[pl.ds(base, DEG_CHB)], idx_v)
        pltpu.sync_copy(ones_v, deg_sh.at[idx_v], add=True)
        return carry

    lax.fori_loop(0, EPW // DEG_CHB, body, 0)
    plsc.subcore_barrier()
    pltpu.sync_copy(deg_sh.at[pl.ds(s * slab, slab)], zv)
    pltpu.sync_copy(zv, out_hbm.at[pl.ds(c * DEG_N + s * slab, slab)])


@functools.partial(
    pl.kernel,
    out_type=jax.ShapeDtypeStruct((T_ROWS, D), jnp.float32),
    mesh=_mesh,
    compiler_params=pltpu.CompilerParams(use_tc_tiling_on_sc=False),
    scratch_types=[
        pltpu.VMEM((2, CHB), jnp.int32),
        pltpu.VMEM((2, CHB), jnp.int32),
        pltpu.VMEM((CHB,), jnp.int32),
        pltpu.VMEM((CHB,), jnp.int32),
        pltpu.VMEM((CHB,), jnp.int32),
        pltpu.VMEM((CHB,), jnp.int32),
        pltpu.VMEM((CHB, D), jnp.float32),
        pltpu.VMEM((CHB, D), jnp.float32),
        pltpu.VMEM_SHARED((ACC_ROWS, D), jnp.float32),
        pltpu.SemaphoreType.DMA,
        pltpu.SemaphoreType.DMA,
        pltpu.SemaphoreType.DMA,
        pltpu.SemaphoreType.DMA,
        pltpu.SemaphoreType.DMA,
        pltpu.SemaphoreType.DMA,
    ],
)
def _scatter_kernel(t_hbm, rc_hbm, s_hbm,
                    ibuf0, ibuf1, gix0, gix1, six0, six1, gbuf0, gbuf1,
                    acc, isem0, isem1, gsem0, gsem1, ssem0, ssem1):
    c = lax.axis_index("c")
    s = lax.axis_index("s")
    slab = ACC_ROWS // NS
    cbase = c * HALF
    ibuf = (ibuf0, ibuf1)
    gix = (gix0, gix1)
    six = (six0, six1)
    gbuf = (gbuf0, gbuf1)
    isem = (isem0, isem1)
    gsem = (gsem0, gsem1)
    ssem = (ssem0, ssem1)

    def zero_fill(dst, dst_off, n):
        full, rem = divmod(n, CHB)
        for r in range(full):
            pltpu.sync_copy(gbuf0, dst.at[pl.ds(dst_off + r * CHB, CHB)])
        if rem:
            pltpu.sync_copy(gbuf0.at[pl.ds(0, rem)],
                            dst.at[pl.ds(dst_off + full * CHB, rem)])

    def zbody(i, carry):
        for j in range(D // 16):
            gbuf0[i, pl.ds(j * 16, 16)] = jnp.zeros((16,), jnp.float32)
        return carry

    lax.fori_loop(0, CHB, zbody, 0)
    zero_fill(acc, s * slab, slab)

    @pl.when((c == NC - 1) & (s == NS - 1))
    def _():
        zero_fill(s_hbm, N_NODES, T_ROWS - N_NODES)

    plsc.subcore_barrier()

    def idx_start(q, j):
        return pltpu.async_copy(rc_hbm.at[s * NCHP + q], ibuf[j], isem[j])

    def idx_wait(q, j):
        pltpu.make_async_copy(rc_hbm.at[s * NCHP + q], ibuf[j],
                              isem[j]).wait()

    def compute_idx(j):
        for i in range(CHB // 16):
            sl = pl.ds(i * 16, 16)
            rv = ibuf[j][0, sl]
            cv = ibuf[j][1, sl]
            local = cv - cbase
            inr = (local >= 0) & (local < HALF)
            alt = jnp.where(local < 0, local + HALF, local - HALF)
            six[j][sl] = jnp.where(inr, local, alt)
            gix[j][sl] = jnp.where(inr, rv, N_NODES + (rv & 1023))

    def gather_start(j):
        pltpu.async_copy(t_hbm.at[gix[j]], gbuf[j], gsem[j])

    def gather_wait(j):
        pltpu.make_async_copy(t_hbm.at[gix[j]], gbuf[j], gsem[j]).wait()

    def scatter_start(j):
        pltpu.async_copy(gbuf[j], acc.at[six[j]], ssem[j], add=True)

    def scatter_wait(j):
        pltpu.make_async_copy(gbuf[j], acc.at[six[j]], ssem[j]).wait()

    idx_start(0, 0)
    idx_start(1, 1)
    idx_wait(0, 0)
    compute_idx(0)
    idx_start(2, 0)
    gather_start(0)
    idx_wait(1, 1)
    compute_idx(1)
    idx_start(3, 1)
    gather_start(1)
    gather_wait(0)
    scatter_start(0)

    def body(p, carry):
        a = 2 * p
        idx_wait(a, 0)
        scatter_wait(0)
        compute_idx(0)
        idx_start(a + 2, 0)
        gather_start(0)
        gather_wait(1)
        scatter_start(1)
        idx_wait(a + 1, 1)
        scatter_wait(1)
        compute_idx(1)
        idx_start(a + 3, 1)
        gather_start(1)
        gather_wait(0)
        scatter_start(0)
        return carry

    lax.fori_loop(1, NCH // 2, body, 0)

    gather_wait(1)
    scatter_start(1)
    scatter_wait(0)
    scatter_wait(1)
    idx_wait(NCH, 0)
    idx_wait(NCH + 1, 1)

    plsc.subcore_barrier()

    def copy_rows(src_off, dst_off, n):
        full, rem = divmod(n, CHB)
        for r in range(full):
            pltpu.sync_copy(acc.at[pl.ds(src_off + r * CHB, CHB)], gbuf0)
            pltpu.sync_copy(gbuf0, s_hbm.at[pl.ds(dst_off + r * CHB, CHB)])
        if rem:
            pltpu.sync_copy(acc.at[pl.ds(src_off + full * CHB, rem)],
                            gbuf0.at[pl.ds(0, rem)])
            pltpu.sync_copy(gbuf0.at[pl.ds(0, rem)],
                            s_hbm.at[pl.ds(dst_off + full * CHB, rem)])

    @pl.when(s < NS - 1)
    def _():
        copy_rows(s * slab, cbase + s * slab, slab)

    @pl.when(s == NS - 1)
    def _():
        last = HALF - (NS - 1) * slab
        copy_rows((NS - 1) * slab, cbase + (NS - 1) * slab, last)


def _dinv_body(pd_ref, out_ref):
    a = pd_ref[0:DEG_N]
    b = pd_ref[DEG_N:2 * DEG_N]
    deg = a + b
    out_ref[...] = jnp.where(deg > 0.0,
                             lax.rsqrt(jnp.maximum(deg, 1e-30)), 0.0)


def _scale_body(e_ref, d_ref, t_ref):
    t_ref[...] = d_ref[...] * e_ref[...]


def _combine_body(s_ref, d_ref, a_ref, acc_ref, t_ref):
    d = d_ref[...]
    e = d * s_ref[...]
    acc_ref[...] = a_ref[...] + e
    t_ref[...] = d * e


def _final_body(s_ref, d_ref, a_ref, out_ref):
    e = d_ref[...] * s_ref[...]
    out_ref[...] = (a_ref[...] + e) * 0.25


_BLK = 1024
_GRID = T_ROWS // _BLK

_row_spec = pl.BlockSpec((_BLK, D), lambda i: (i, 0))
_col_spec = pl.BlockSpec((_BLK, 1), lambda i: (i, 0))
_f32t = jax.ShapeDtypeStruct((T_ROWS, D), jnp.float32)

_dinv_call = pl.pallas_call(
    _dinv_body,
    out_shape=jax.ShapeDtypeStruct((DEG_N,), jnp.float32),
)

_scale_call = pl.pallas_call(
    _scale_body,
    grid=(_GRID,),
    in_specs=[_row_spec, _col_spec],
    out_specs=_row_spec,
    out_shape=_f32t,
)

_combine_call = pl.pallas_call(
    _combine_body,
    grid=(_GRID,),
    in_specs=[_row_spec, _col_spec, _row_spec],
    out_specs=(_row_spec, _row_spec),
    out_shape=(_f32t, _f32t),
)

_final_call = pl.pallas_call(
    _final_body,
    grid=(_GRID,),
    in_specs=[_row_spec, _col_spec, _row_spec],
    out_specs=_row_spec,
    out_shape=_f32t,
)


def kernel(edge_index, user_emb, item_emb):
    row = edge_index[0].astype(jnp.int32)
    col = edge_index[1].astype(jnp.int32)
    npad = E_PAD - E
    pad_i = jnp.arange(npad, dtype=jnp.int32)
    row_p = jnp.concatenate([row, N_NODES + (pad_i % 1024)])
    col_p = jnp.concatenate([col, pad_i % HALF])

    row_r = row_p.reshape(NS, NCH, 1, CHB)
    col_r = col_p.reshape(NS, NCH, 1, CHB)
    rc = jnp.concatenate([row_r, col_r], axis=2)
    rc = jnp.concatenate(
        [rc, jnp.zeros((NS, NCHP - NCH, 2, CHB), jnp.int32)], axis=1)
    rc = rc.reshape(NS * NCHP, 2, CHB)

    deg2 = _deg_kernel(row_p)
    dinv = _dinv_call(deg2)

    dinv_col = jnp.concatenate(
        [dinv[:N_NODES], jnp.zeros((T_ROWS - N_NODES,), jnp.float32)]
    ).reshape(T_ROWS, 1)
    e0 = jnp.concatenate(
        [user_emb, item_emb,
         jnp.zeros((T_ROWS - N_NODES, D), jnp.float32)], axis=0)

    t = _scale_call(e0, dinv_col)
    acc = e0
    for layer in range(N_LAYERS):
        sarr = _scatter_kernel(t, rc)
        if layer < N_LAYERS - 1:
            acc, t = _combine_call(sarr, dinv_col, acc)
        else:
            acc = _final_call(sarr, dinv_col, acc)
    return acc[:N_NODES]


# speedup vs baseline: 17.9437x; 1.0485x over previous
import functools

import jax
import jax.numpy as jnp
from jax import lax
from jax.experimental import pallas as pl
from jax.experimental.pallas import tpu as pltpu
from jax.experimental.pallas import tpu_sc as plsc

N_USERS = 10000
N_ITEMS = 40000
N_NODES = 50000
D = 64
E = 800000
N_LAYERS = 3

NC = 2
NS = 16
HALF = N_NODES // NC
ACC_ROWS = NS * 1568
T_ROWS = 51200
DEG_N = NS * 3200

CHB = 224
NCH = 224
NCHP = NCH + 2
EPT = NCH * CHB
E_PAD = NS * EPT
DEG_CHB = 512
EPW = E_PAD // (NC * NS)

_mesh = plsc.VectorSubcoreMesh(core_axis_name="c", subcore_axis_name="s")


@functools.partial(
    pl.kernel,
    out_type=jax.ShapeDtypeStruct((2 * DEG_N,), jnp.float32),
    mesh=_mesh,
    compiler_params=pltpu.CompilerParams(use_tc_tiling_on_sc=False),
    scratch_types=[
        pltpu.VMEM((DEG_CHB,), jnp.int32),
        pltpu.VMEM((DEG_CHB,), jnp.float32),
        pltpu.VMEM((DEG_N // NS,), jnp.float32),
        pltpu.VMEM_SHARED((DEG_N,), jnp.float32),
    ],
)
def _deg_kernel(row_hbm, out_hbm, idx_v, ones_v, zv, deg_sh):
    c = lax.axis_index("c")
    s = lax.axis_index("s")
    wid = c * NS + s
    slab = DEG_N // NS
    for i in range(DEG_CHB // 16):
        ones_v[pl.ds(i * 16, 16)] = jnp.full((16,), 1.0, jnp.float32)

    def zbody(i, carry):
        zv[pl.ds(i * 16, 16)] = jnp.zeros((16,), jnp.float32)
        return carry

    lax.fori_loop(0, slab // 16, zbody, 0)
    pltpu.sync_copy(zv, deg_sh.at[pl.ds(s * slab, slab)])
    plsc.subcore_barrier()

    def body(k, carry):
        base = wid * EPW + k * DEG_CHB
        pltpu.sync_copy(row_hbm.at[pl.ds(base, DEG_CHB)], idx_v)
        pltpu.sync_copy(ones_v, deg_sh.at[idx_v], add=True)
        return carry

    lax.fori_loop(0, EPW // DEG_CHB, body, 0)
    plsc.subcore_barrier()
    pltpu.sync_copy(deg_sh.at[pl.ds(s * slab, slab)], zv)
    pltpu.sync_copy(zv, out_hbm.at[pl.ds(c * DEG_N + s * slab, slab)])


def _make_layer_kernel(is_last):
    if is_last:
        out_type = jax.ShapeDtypeStruct((T_ROWS, D), jnp.float32)
    else:
        out_type = (jax.ShapeDtypeStruct((T_ROWS, D), jnp.float32),
                    jax.ShapeDtypeStruct((T_ROWS, D), jnp.float32))

    @functools.partial(
        pl.kernel,
        out_type=out_type,
        mesh=_mesh,
        compiler_params=pltpu.CompilerParams(use_tc_tiling_on_sc=False),
        scratch_types=[
            pltpu.VMEM((2, CHB), jnp.int32),
            pltpu.VMEM((2, CHB), jnp.int32),
            pltpu.VMEM((CHB,), jnp.int32),
            pltpu.VMEM((CHB,), jnp.int32),
            pltpu.VMEM((CHB,), jnp.int32),
            pltpu.VMEM((CHB,), jnp.int32),
            pltpu.VMEM((CHB, D), jnp.float32),
            pltpu.VMEM((CHB, D), jnp.float32),
            pltpu.VMEM_SHARED((ACC_ROWS, D), jnp.float32),
            pltpu.SemaphoreType.DMA,
            pltpu.SemaphoreType.DMA,
            pltpu.SemaphoreType.DMA,
            pltpu.SemaphoreType.DMA,
            pltpu.SemaphoreType.DMA,
            pltpu.SemaphoreType.DMA,
        ],
    )
    def _layer(t_hbm, rc_hbm, de_hbm, pp_hbm, *rest):
        if is_last:
            (p_hbm, ibuf0, ibuf1, gix0, gix1, six0, six1, gbuf0, gbuf1,
             acc, isem0, isem1, gsem0, gsem1, ssem0, ssem1) = rest
            t_out = None
        else:
            (t_out, p_hbm, ibuf0, ibuf1, gix0, gix1, six0, six1, gbuf0,
             gbuf1, acc, isem0, isem1, gsem0, gsem1, ssem0, ssem1) = rest
        c = lax.axis_index("c")
        s = lax.axis_index("s")
        slab = ACC_ROWS // NS
        cbase = c * HALF
        ibuf = (ibuf0, ibuf1)
        gix = (gix0, gix1)
        six = (six0, six1)
        gbuf = (gbuf0, gbuf1)
        isem = (isem0, isem1)
        gsem = (gsem0, gsem1)
        ssem = (ssem0, ssem1)

        def zero_fill(dst, dst_off, n):
            full, rem = divmod(n, CHB)
            for r in range(full):
                pltpu.sync_copy(gbuf0, dst.at[pl.ds(dst_off + r * CHB, CHB)])
            if rem:
                pltpu.sync_copy(gbuf0.at[pl.ds(0, rem)],
                                dst.at[pl.ds(dst_off + full * CHB, rem)])

        def zbody(i, carry):
            for j in range(D // 16):
                gbuf0[i, pl.ds(j * 16, 16)] = jnp.zeros((16,), jnp.float32)
            return carry

        lax.fori_loop(0, CHB, zbody, 0)
        zero_fill(acc, s * slab, slab)

        @pl.when((c == NC - 1) & (s == NS - 1))
        def _():
            zero_fill(p_hbm, N_NODES, T_ROWS - N_NODES)
            if not is_last:
                zero_fill(t_out, N_NODES, T_ROWS - N_NODES)

        plsc.subcore_barrier()

        def idx_start(q, j):
            pltpu.async_copy(rc_hbm.at[s * NCHP + q], ibuf[j], isem[j])

        def idx_wait(q, j):
            pltpu.make_async_copy(rc_hbm.at[s * NCHP + q], ibuf[j],
                                  isem[j]).wait()

        def compute_idx(j):
            for i in range(CHB // 16):
                sl = pl.ds(i * 16, 16)
                rv = ibuf[j][0, sl]
                cv = ibuf[j][1, sl]
                local = cv - cbase
                inr = (local >= 0) & (local < HALF)
                alt = jnp.where(local < 0, local + HALF, local - HALF)
                six[j][sl] = jnp.where(inr, local, alt)
                gix[j][sl] = jnp.where(inr, rv, N_NODES + (rv & 1023))

        def gather_start(j):
            pltpu.async_copy(t_hbm.at[gix[j]], gbuf[j], gsem[j])

        def gather_wait(j):
            pltpu.make_async_copy(t_hbm.at[gix[j]], gbuf[j], gsem[j]).wait()

        def scatter_start(j):
            pltpu.async_copy(gbuf[j], acc.at[six[j]], ssem[j], add=True)

        def scatter_wait(j):
            pltpu.make_async_copy(gbuf[j], acc.at[six[j]], ssem[j]).wait()

        idx_start(0, 0)
        idx_start(1, 1)
        idx_wait(0, 0)
        compute_idx(0)
        idx_start(2, 0)
        gather_start(0)
        idx_wait(1, 1)
        compute_idx(1)
        idx_start(3, 1)
        gather_start(1)
        gather_wait(0)
        scatter_start(0)

        def body(p, carry):
            a = 2 * p
            idx_wait(a, 0)
            scatter_wait(0)
            compute_idx(0)
            idx_start(a + 2, 0)
            gather_start(0)
            gather_wait(1)
            scatter_start(1)
            idx_wait(a + 1, 1)
            scatter_wait(1)
            compute_idx(1)
            idx_start(a + 3, 1)
            gather_start(1)
            gather_wait(0)
            scatter_start(0)
            return carry

        lax.fori_loop(1, NCH // 2, body, 0)

        gather_wait(1)
        scatter_start(1)
        scatter_wait(0)
        scatter_wait(1)
        idx_wait(NCH, 0)
        idx_wait(NCH + 1, 1)

        plsc.subcore_barrier()

        def copy_rows(src_off, n):
            full, rem = divmod(n, CHB)
            sizes = [CHB] * full + ([rem] if rem else [])
            for r, nn in enumerate(sizes):
                lo = src_off + r * CHB
                glo = cbase + lo
                pltpu.sync_copy(acc.at[pl.ds(lo, nn)],
                                gbuf0.at[pl.ds(0, nn)])
                pltpu.sync_copy(de_hbm.at[pl.ds(glo, nn)],
                                gbuf1.at[pl.ds(0, nn)])

                def scale_row(i, carry):
                    for j in range(D // 16):
                        sl = pl.ds(j * 16, 16)
                        a = gbuf0[i, sl]
                        d = gbuf1[i, sl]
                        e = a * d
                        if not is_last:
                            gbuf0[i, sl] = d * e
                        gbuf1[i, sl] = e
                    return carry

                lax.fori_loop(0, nn, scale_row, 0)
                if not is_last:
                    pltpu.sync_copy(gbuf0.at[pl.ds(0, nn)],
                                    t_out.at[pl.ds(glo, nn)])
                pltpu.sync_copy(pp_hbm.at[pl.ds(glo, nn)],
                                gbuf0.at[pl.ds(0, nn)])

                def add_row(i, carry):
                    for j in range(D // 16):
                        sl = pl.ds(j * 16, 16)
                        x = gbuf0[i, sl] + gbuf1[i, sl]
                        if is_last:
                            x = x * 0.25
                        gbuf0[i, sl] = x
                    return carry

                lax.fori_loop(0, nn, add_row, 0)
                pltpu.sync_copy(gbuf0.at[pl.ds(0, nn)],
                                p_hbm.at[pl.ds(glo, nn)])

        @pl.when(s < NS - 1)
        def _():
            copy_rows(s * slab, slab)

        @pl.when(s == NS - 1)
        def _():
            copy_rows((NS - 1) * slab, HALF - (NS - 1) * slab)

    return _layer


_layer_mid = _make_layer_kernel(is_last=False)
_layer_last = _make_layer_kernel(is_last=True)


_BLK = 1024
_GRID = T_ROWS // _BLK


def _a2_body(d0_ref, d1_ref, e_ref, dex_ref, t_ref):
    deg = d0_ref[...] + d1_ref[...]
    dc = jnp.where(deg > 0.0, lax.rsqrt(jnp.maximum(deg, 1e-30)), 0.0)
    dex_ref[...] = dc * jnp.ones((_BLK, D), jnp.float32)
    t_ref[...] = dc * e_ref[...]


_row_spec = pl.BlockSpec((_BLK, D), lambda i: (i, 0))
_col_spec = pl.BlockSpec((_BLK, 1), lambda i: (i, 0))
_f32t = jax.ShapeDtypeStruct((T_ROWS, D), jnp.float32)

_a2_call = pl.pallas_call(
    _a2_body,
    grid=(_GRID,),
    in_specs=[_col_spec, _col_spec, _row_spec],
    out_specs=(_row_spec, _row_spec),
    out_shape=(_f32t, _f32t),
)


def kernel(edge_index, user_emb, item_emb):
    row = edge_index[0].astype(jnp.int32)
    col = edge_index[1].astype(jnp.int32)
    npad = E_PAD - E
    pad_i = jnp.arange(npad, dtype=jnp.int32)
    row_p = jnp.concatenate([row, N_NODES + (pad_i % 1024)])
    col_p = jnp.concatenate([col, pad_i % HALF])

    row_r = row_p.reshape(NS, NCH, 1, CHB)
    col_r = col_p.reshape(NS, NCH, 1, CHB)
    rc = jnp.concatenate([row_r, col_r], axis=2)
    rc = jnp.concatenate(
        [rc, jnp.zeros((NS, NCHP - NCH, 2, CHB), jnp.int32)], axis=1)
    rc = rc.reshape(NS * NCHP, 2, CHB)

    deg2 = _deg_kernel(row_p)
    d0 = deg2[:T_ROWS].reshape(T_ROWS, 1)
    d1 = deg2[T_ROWS:].reshape(T_ROWS, 1)
    e0 = jnp.concatenate(
        [user_emb, item_emb,
         jnp.zeros((T_ROWS - N_NODES, D), jnp.float32)], axis=0)
    dex, t = _a2_call(d0, d1, e0)

    p = e0
    for layer in range(N_LAYERS):
        if layer < N_LAYERS - 1:
            t, p = _layer_mid(t, rc, dex, p)
        else:
            p = _layer_last(t, rc, dex, p)
    return p[:N_NODES]


# speedup vs baseline: 17.9937x; 1.0028x over previous
import functools

import jax
import jax.numpy as jnp
from jax import lax
from jax.experimental import pallas as pl
from jax.experimental.pallas import tpu as pltpu
from jax.experimental.pallas import tpu_sc as plsc

N_USERS = 10000
N_ITEMS = 40000
N_NODES = 50000
D = 64
E = 800000
N_LAYERS = 3

NC = 2
NS = 16
HALF = N_NODES // NC
ACC_ROWS = NS * 1568
T_ROWS = 51200
DEG_N = NS * 3200

CHB = 224
NCH = 224
NCHP = NCH + 2
EPT = NCH * CHB
E_PAD = NS * EPT
DEG_CHB = 512
EPW = E_PAD // (NC * NS)

_mesh = plsc.VectorSubcoreMesh(core_axis_name="c", subcore_axis_name="s")


@functools.partial(
    pl.kernel,
    out_type=jax.ShapeDtypeStruct((2 * DEG_N,), jnp.float32),
    mesh=_mesh,
    compiler_params=pltpu.CompilerParams(use_tc_tiling_on_sc=False),
    scratch_types=[
        pltpu.VMEM((DEG_CHB,), jnp.int32),
        pltpu.VMEM((DEG_CHB,), jnp.float32),
        pltpu.VMEM((DEG_N // NS,), jnp.float32),
        pltpu.VMEM_SHARED((DEG_N,), jnp.float32),
    ],
)
def _deg_kernel(row_hbm, out_hbm, idx_v, ones_v, zv, deg_sh):
    c = lax.axis_index("c")
    s = lax.axis_index("s")
    wid = c * NS + s
    slab = DEG_N // NS
    for i in range(DEG_CHB // 16):
        ones_v[pl.ds(i * 16, 16)] = jnp.full((16,), 1.0, jnp.float32)

    def zbody(i, carry):
        zv[pl.ds(i * 16, 16)] = jnp.zeros((16,), jnp.float32)
        return carry

    lax.fori_loop(0, slab // 16, zbody, 0)
    pltpu.sync_copy(zv, deg_sh.at[pl.ds(s * slab, slab)])
    plsc.subcore_barrier()

    def body(k, carry):
        base = wid * EPW + k * DEG_CHB
        pltpu.sync_copy(row_hbm.at[pl.ds(base, DEG_CHB)], idx_v)
        pltpu.sync_copy(ones_v, deg_sh.at[idx_v], add=True)
        return carry

    lax.fori_loop(0, EPW // DEG_CHB, body, 0)
    plsc.subcore_barrier()
    pltpu.sync_copy(deg_sh.at[pl.ds(s * slab, slab)], zv)
    pltpu.sync_copy(zv, out_hbm.at[pl.ds(c * DEG_N + s * slab, slab)])


def _make_layer_kernel(is_last):
    if is_last:
        out_type = jax.ShapeDtypeStruct((T_ROWS, D), jnp.float32)
    else:
        out_type = (jax.ShapeDtypeStruct((T_ROWS, D), jnp.float32),
                    jax.ShapeDtypeStruct((T_ROWS, D), jnp.float32))

    @functools.partial(
        pl.kernel,
        out_type=out_type,
        mesh=_mesh,
        compiler_params=pltpu.CompilerParams(use_tc_tiling_on_sc=False),
        scratch_types=[
            pltpu.VMEM((2, CHB), jnp.int32),
            pltpu.VMEM((2, CHB), jnp.int32),
            pltpu.VMEM((CHB,), jnp.int32),
            pltpu.VMEM((CHB,), jnp.int32),
            pltpu.VMEM((CHB,), jnp.int32),
            pltpu.VMEM((CHB,), jnp.int32),
            pltpu.VMEM((CHB, D), jnp.float32),
            pltpu.VMEM((CHB, D), jnp.float32),
            pltpu.VMEM_SHARED((ACC_ROWS, D), jnp.float32),
            pltpu.SemaphoreType.DMA,
            pltpu.SemaphoreType.DMA,
            pltpu.SemaphoreType.DMA,
            pltpu.SemaphoreType.DMA,
            pltpu.SemaphoreType.DMA,
            pltpu.SemaphoreType.DMA,
        ],
    )
    def _layer(t_hbm, rc_hbm, de_hbm, pp_hbm, *rest):
        if is_last:
            (p_hbm, ibuf0, ibuf1, gix0, gix1, six0, six1, gbuf0, gbuf1,
             acc, isem0, isem1, gsem0, gsem1, ssem0, ssem1) = rest
            t_out = None
        else:
            (t_out, p_hbm, ibuf0, ibuf1, gix0, gix1, six0, six1, gbuf0,
             gbuf1, acc, isem0, isem1, gsem0, gsem1, ssem0, ssem1) = rest
        c = lax.axis_index("c")
        s = lax.axis_index("s")
        slab = ACC_ROWS // NS
        cbase = c * HALF
        ibuf = (ibuf0, ibuf1)
        gix = (gix0, gix1)
        six = (six0, six1)
        gbuf = (gbuf0, gbuf1)
        isem = (isem0, isem1)
        gsem = (gsem0, gsem1)
        ssem = (ssem0, ssem1)

        pltpu.async_copy(rc_hbm.at[s * NCHP + 0], ibuf0, isem0)
        pltpu.async_copy(rc_hbm.at[s * NCHP + 1], ibuf1, isem1)

        def zero_fill(dst, dst_off, n):
            full, rem = divmod(n, CHB)
            for r in range(full):
                pltpu.async_copy(gbuf0, dst.at[pl.ds(dst_off + r * CHB, CHB)],
                                 gsem0)
            if rem:
                pltpu.async_copy(gbuf0.at[pl.ds(0, rem)],
                                 dst.at[pl.ds(dst_off + full * CHB, rem)],
                                 gsem0)
            for r in range(full):
                pltpu.make_async_copy(
                    gbuf0, dst.at[pl.ds(dst_off + r * CHB, CHB)],
                    gsem0).wait()
            if rem:
                pltpu.make_async_copy(
                    gbuf0.at[pl.ds(0, rem)],
                    dst.at[pl.ds(dst_off + full * CHB, rem)], gsem0).wait()

        def zbody(i, carry):
            for j in range(D // 16):
                gbuf0[i, pl.ds(j * 16, 16)] = jnp.zeros((16,), jnp.float32)
            return carry

        lax.fori_loop(0, CHB, zbody, 0)
        zero_fill(acc, s * slab, slab)

        @pl.when((c == NC - 1) & (s == NS - 1))
        def _():
            zero_fill(p_hbm, N_NODES, T_ROWS - N_NODES)
            if not is_last:
                zero_fill(t_out, N_NODES, T_ROWS - N_NODES)

        plsc.subcore_barrier()

        def idx_start(q, j):
            pltpu.async_copy(rc_hbm.at[s * NCHP + q], ibuf[j], isem[j])

        def idx_wait(q, j):
            pltpu.make_async_copy(rc_hbm.at[s * NCHP + q], ibuf[j],
                                  isem[j]).wait()

        def compute_idx(j):
            for i in range(CHB // 16):
                sl = pl.ds(i * 16, 16)
                rv = ibuf[j][0, sl]
                cv = ibuf[j][1, sl]
                local = cv - cbase
                inr = (local >= 0) & (local < HALF)
                alt = jnp.where(local < 0, local + HALF, local - HALF)
                six[j][sl] = jnp.where(inr, local, alt)
                gix[j][sl] = jnp.where(inr, rv, N_NODES + (rv & 1023))

        def gather_start(j):
            pltpu.async_copy(t_hbm.at[gix[j]], gbuf[j], gsem[j])

        def gather_wait(j):
            pltpu.make_async_copy(t_hbm.at[gix[j]], gbuf[j], gsem[j]).wait()

        def scatter_start(j):
            pltpu.async_copy(gbuf[j], acc.at[six[j]], ssem[j], add=True)

        def scatter_wait(j):
            pltpu.make_async_copy(gbuf[j], acc.at[six[j]], ssem[j]).wait()

        idx_wait(0, 0)
        compute_idx(0)
        idx_start(2, 0)
        gather_start(0)
        idx_wait(1, 1)
        compute_idx(1)
        idx_start(3, 1)
        gather_start(1)
        gather_wait(0)
        scatter_start(0)

        def body(p, carry):
            a = 2 * p
            idx_wait(a, 0)
            scatter_wait(0)
            compute_idx(0)
            idx_start(a + 2, 0)
            gather_start(0)
            gather_wait(1)
            scatter_start(1)
            idx_wait(a + 1, 1)
            scatter_wait(1)
            compute_idx(1)
            idx_start(a + 3, 1)
            gather_start(1)
            gather_wait(0)
            scatter_start(0)
            return carry

        lax.fori_loop(1, NCH // 2, body, 0)

        gather_wait(1)
        scatter_start(1)
        scatter_wait(0)
        scatter_wait(1)
        idx_wait(NCH, 0)
        idx_wait(NCH + 1, 1)

        plsc.subcore_barrier()

        def copy_rows(src_off, n):
            full, rem = divmod(n, CHB)
            sizes = [CHB] * full + ([rem] if rem else [])
            for r, nn in enumerate(sizes):
                lo = src_off + r * CHB
                glo = cbase + lo
                pltpu.sync_copy(acc.at[pl.ds(lo, nn)],
                                gbuf0.at[pl.ds(0, nn)])
                pltpu.sync_copy(de_hbm.at[pl.ds(glo, nn)],
                                gbuf1.at[pl.ds(0, nn)])

                def scale_row(i, carry):
                    for j in range(D // 16):
                        sl = pl.ds(j * 16, 16)
                        a = gbuf0[i, sl]
                        d = gbuf1[i, sl]
                        e = a * d
                        if not is_last:
                            gbuf0[i, sl] = d * e
                        gbuf1[i, sl] = e
                    return carry

                lax.fori_loop(0, nn, scale_row, 0)
                if not is_last:
                    pltpu.sync_copy(gbuf0.at[pl.ds(0, nn)],
                                    t_out.at[pl.ds(glo, nn)])
                pltpu.sync_copy(pp_hbm.at[pl.ds(glo, nn)],
                                gbuf0.at[pl.ds(0, nn)])

                def add_row(i, carry):
                    for j in range(D // 16):
                        sl = pl.ds(j * 16, 16)
                        x = gbuf0[i, sl] + gbuf1[i, sl]
                        if is_last:
                            x = x * 0.25
                        gbuf0[i, sl] = x
                    return carry

                lax.fori_loop(0, nn, add_row, 0)
                pltpu.sync_copy(gbuf0.at[pl.ds(0, nn)],
                                p_hbm.at[pl.ds(glo, nn)])

        @pl.when(s < NS - 1)
        def _():
            copy_rows(s * slab, slab)

        @pl.when(s == NS - 1)
        def _():
            copy_rows((NS - 1) * slab, HALF - (NS - 1) * slab)

    return _layer


_layer_mid = _make_layer_kernel(is_last=False)
_layer_last = _make_layer_kernel(is_last=True)


_BLK = 1024
_GRID = T_ROWS // _BLK


def _a2_body(d0_ref, d1_ref, e_ref, dex_ref, t_ref):
    deg = d0_ref[...] + d1_ref[...]
    dc = jnp.where(deg > 0.0, lax.rsqrt(jnp.maximum(deg, 1e-30)), 0.0)
    dex_ref[...] = dc * jnp.ones((_BLK, D), jnp.float32)
    t_ref[...] = dc * e_ref[...]


_row_spec = pl.BlockSpec((_BLK, D), lambda i: (i, 0))
_col_spec = pl.BlockSpec((_BLK, 1), lambda i: (i, 0))
_f32t = jax.ShapeDtypeStruct((T_ROWS, D), jnp.float32)

_a2_call = pl.pallas_call(
    _a2_body,
    grid=(_GRID,),
    in_specs=[_col_spec, _col_spec, _row_spec],
    out_specs=(_row_spec, _row_spec),
    out_shape=(_f32t, _f32t),
)


def kernel(edge_index, user_emb, item_emb):
    row = edge_index[0].astype(jnp.int32)
    col = edge_index[1].astype(jnp.int32)
    npad = E_PAD - E
    pad_i = jnp.arange(npad, dtype=jnp.int32)
    row_p = jnp.concatenate([row, N_NODES + (pad_i % 1024)])
    col_p = jnp.concatenate([col, pad_i % HALF])

    row_r = row_p.reshape(NS, NCH, 1, CHB)
    col_r = col_p.reshape(NS, NCH, 1, CHB)
    rc = jnp.concatenate([row_r, col_r], axis=2)
    rc = jnp.concatenate(
        [rc, jnp.zeros((NS, NCHP - NCH, 2, CHB), jnp.int32)], axis=1)
    rc = rc.reshape(NS * NCHP, 2, CHB)

    deg2 = _deg_kernel(row_p)
    d0 = deg2[:T_ROWS].reshape(T_ROWS, 1)
    d1 = deg2[T_ROWS:].reshape(T_ROWS, 1)
    e0 = jnp.concatenate(
        [user_emb, item_emb,
         jnp.zeros((T_ROWS - N_NODES, D), jnp.float32)], axis=0)
    dex, t = _a2_call(d0, d1, e0)

    p = e0
    for layer in range(N_LAYERS):
        if layer < N_LAYERS - 1:
            t, p = _layer_mid(t, rc, dex, p)
        else:
            p = _layer_last(t, rc, dex, p)
    return p[:N_NODES]
